# baseline (speedup 1.0000x reference)
import jax
import jax.numpy as jnp
from jax.experimental import pallas as pl
from jax.experimental.pallas import tpu as pltpu


def _copy_all_kernel(pdd_in, pvd_in, pdr_in, pvr_in, cell_in,
                     pdd_out, pvd_out, pdr_out, pvr_out, cell_out,
                     *sems):
    copies = [
        pltpu.make_async_copy(pdd_in, pdd_out, sems[0]),
        pltpu.make_async_copy(pvd_in, pvd_out, sems[1]),
        pltpu.make_async_copy(pdr_in, pdr_out, sems[2]),
        pltpu.make_async_copy(pvr_in, pvr_out, sems[3]),
        pltpu.make_async_copy(cell_in, cell_out, sems[4]),
    ]
    for c in copies:
        c.start()
    for c in copies:
        c.wait()


def kernel(natoms, pred_distance_displace, pred_var_displace,
           pred_distance_relaxed, pred_var_relaxed, pred_cell):
    any_spec = pl.BlockSpec(memory_space=pl.ANY)
    big2d = (2032, 1024)
    a = pred_distance_displace.reshape(big2d)
    b = pred_var_displace.reshape(big2d)
    c = pred_distance_relaxed.reshape(big2d)
    d = pred_var_relaxed.reshape(big2d)
    outs = pl.pallas_call(
        _copy_all_kernel,
        in_specs=[any_spec] * 5,
        out_specs=[any_spec] * 5,
        out_shape=[
            jax.ShapeDtypeStruct(big2d, jnp.float32),
            jax.ShapeDtypeStruct(big2d, jnp.float32),
            jax.ShapeDtypeStruct(big2d, jnp.float32),
            jax.ShapeDtypeStruct(big2d, jnp.float32),
            jax.ShapeDtypeStruct(pred_cell.shape, jnp.float32),
        ],
        scratch_shapes=[pltpu.SemaphoreType.DMA] * 5,
    )(a, b, c, d, pred_cell)
    n = pred_distance_displace.shape[0]
    return (outs[0].reshape(n), outs[1].reshape(n), outs[2].reshape(n),
            outs[3].reshape(n), outs[4])


# speedup vs baseline: 10.2883x; 10.2883x over previous
import jax
import jax.numpy as jnp
from jax.experimental import pallas as pl
from jax.experimental.pallas import tpu as pltpu

_TOTAL = 128 * 128 * 127
_ROWS, _LANES = 127, 16384
_GRID = 8
_BLK = _LANES // _GRID


def _copy_kernel(a_in, b_in, c_in, d_in, cell_in,
                 a_out, b_out, c_out, d_out, cell_out):
    a_out[...] = a_in[...]
    b_out[...] = b_in[...]
    c_out[...] = c_in[...]
    d_out[...] = d_in[...]

    @pl.when(pl.program_id(0) == 0)
    def _():
        cell_out[...] = cell_in[...]


def kernel(natoms, pred_distance_displace, pred_var_displace,
           pred_distance_relaxed, pred_var_relaxed, pred_cell):
    big_spec = pl.BlockSpec((_ROWS, _BLK), lambda i: (0, i))
    cell_spec = pl.BlockSpec((128, 9), lambda i: (0, 0))
    big_shape = jax.ShapeDtypeStruct((_ROWS, _LANES), jnp.float32)

    a = pred_distance_displace.reshape(_ROWS, _LANES)
    b = pred_var_displace.reshape(_ROWS, _LANES)
    c = pred_distance_relaxed.reshape(_ROWS, _LANES)
    d = pred_var_relaxed.reshape(_ROWS, _LANES)
    cell2d = pred_cell.reshape(128, 9)

    outs = pl.pallas_call(
        _copy_kernel,
        grid=(_GRID,),
        in_specs=[big_spec] * 4 + [cell_spec],
        out_specs=[big_spec] * 4 + [cell_spec],
        out_shape=[big_shape] * 4 + [jax.ShapeDtypeStruct((128, 9), jnp.float32)],
    )(a, b, c, d, cell2d)

    n = pred_distance_displace.shape[0]
    return (outs[0].reshape(n), outs[1].reshape(n), outs[2].reshape(n),
            outs[3].reshape(n), outs[4].reshape(128, 3, 3))


# speedup vs baseline: 42.7457x; 4.1548x over previous
import jax
import jax.numpy as jnp
from jax.experimental import pallas as pl
from jax.experimental.pallas import tpu as pltpu

_TOTAL = 128 * 128 * 127
_GRID = 8
_SUB, _LN = 2032, 128


def _copy_kernel(a_in, b_in, c_in, d_in, cell_in,
                 a_out, b_out, c_out, d_out, cell_out):
    a_out[...] = a_in[...]
    b_out[...] = b_in[...]
    c_out[...] = c_in[...]
    d_out[...] = d_in[...]

    @pl.when(pl.program_id(0) == 0)
    def _():
        cell_out[...] = cell_in[...]


def kernel(natoms, pred_distance_displace, pred_var_displace,
           pred_distance_relaxed, pred_var_relaxed, pred_cell):
    big_spec = pl.BlockSpec((1, _SUB, _LN), lambda i: (i, 0, 0))
    cell_spec = pl.BlockSpec((128, 9), lambda i: (0, 0))
    big_shape = jax.ShapeDtypeStruct((_GRID, _SUB, _LN), jnp.float32)

    a = pred_distance_displace.reshape(_GRID, _SUB, _LN)
    b = pred_var_displace.reshape(_GRID, _SUB, _LN)
    c = pred_distance_relaxed.reshape(_GRID, _SUB, _LN)
    d = pred_var_relaxed.reshape(_GRID, _SUB, _LN)
    cell2d = pred_cell.reshape(128, 9)

    outs = pl.pallas_call(
        _copy_kernel,
        grid=(_GRID,),
        in_specs=[big_spec] * 4 + [cell_spec],
        out_specs=[big_spec] * 4 + [cell_spec],
        out_shape=[big_shape] * 4 + [jax.ShapeDtypeStruct((128, 9), jnp.float32)],
    )(a, b, c, d, cell2d)

    n = pred_distance_displace.shape[0]
    return (outs[0].reshape(n), outs[1].reshape(n), outs[2].reshape(n),
            outs[3].reshape(n), outs[4].reshape(128, 3, 3))
